# baseline (speedup 1.0000x reference)
import functools

import jax
import jax.numpy as jnp
from jax import lax
from jax.experimental import pallas as pl
from jax.experimental.pallas import tpu as pltpu


def _fps_body(x_ref, out_ref, acc_ref, *, b, n, s):
    x0 = x_ref[0]
    x1 = x_ref[1]
    x2 = x_ref[2]
    iota_n = lax.broadcasted_iota(jnp.int32, (1, n), 1)
    iota_s = lax.broadcasted_iota(jnp.int32, (1, s), 1)
    acc_ref[...] = jnp.zeros((3, b, s), jnp.float32)

    def step(t, carry):
        dist, far = carry
        onehot = iota_n == far
        c0 = jnp.sum(jnp.where(onehot, x0, 0.0), axis=1, keepdims=True)
        c1 = jnp.sum(jnp.where(onehot, x1, 0.0), axis=1, keepdims=True)
        c2 = jnp.sum(jnp.where(onehot, x2, 0.0), axis=1, keepdims=True)
        sel = (iota_s == t).astype(jnp.float32)
        acc_ref[0] += c0 * sel
        acc_ref[1] += c1 * sel
        acc_ref[2] += c2 * sel
        d = (x0 - c0) ** 2 + (x1 - c1) ** 2 + (x2 - c2) ** 2
        dist = jnp.minimum(dist, d)
        far = jnp.argmax(dist, axis=1).astype(jnp.int32).reshape(b, 1)
        return dist, far

    init = (jnp.full((b, n), 1e10, jnp.float32), jnp.zeros((b, 1), jnp.int32))
    lax.fori_loop(0, s, step, init)
    out_ref[...] = acc_ref[...]


def _fps_pallas(xyz, s):
    b, n, _ = xyz.shape
    xyzt = jnp.transpose(xyz, (2, 0, 1))
    out = pl.pallas_call(
        functools.partial(_fps_body, b=b, n=n, s=s),
        in_specs=[pl.BlockSpec((3, b, n), lambda: (0, 0, 0))],
        out_specs=pl.BlockSpec((3, b, s), lambda: (0, 0, 0)),
        out_shape=jax.ShapeDtypeStruct((3, b, s), jnp.float32),
        scratch_shapes=[pltpu.VMEM((3, b, s), jnp.float32)],
    )(xyzt)
    return jnp.transpose(out, (1, 2, 0))


def _group_body(idx_ref, p_ref, out_ref, *, n, rows, c_in):
    idx = idx_ref[0]
    iota_n = lax.broadcasted_iota(jnp.int32, (rows, n), 1)
    m = (idx == iota_n).astype(jnp.float32)
    p = p_ref[0]
    hi = p.astype(jnp.bfloat16).astype(jnp.float32)
    r1 = p - hi
    mid = r1.astype(jnp.bfloat16).astype(jnp.float32)
    lo = r1 - mid
    g = jnp.zeros((rows, c_in), jnp.float32)
    for part in (lo, mid, hi):
        g = g + jnp.dot(m, part, preferred_element_type=jnp.float32)
    out_ref[0] = g


def _group_pallas(gidx, nsample, new_xyz, pfeat):
    b, n, c_in = pfeat.shape
    s = new_xyz.shape[1]
    bs = 8
    rows = bs * nsample
    idx_col = gidx.reshape(b, s * nsample, 1)
    out = pl.pallas_call(
        functools.partial(_group_body, n=n, rows=rows, c_in=c_in),
        grid=(b, s // bs),
        in_specs=[
            pl.BlockSpec((1, rows, 1), lambda i, j: (i, j, 0)),
            pl.BlockSpec((1, n, c_in), lambda i, j: (i, 0, 0)),
        ],
        out_specs=pl.BlockSpec((1, rows, c_in), lambda i, j: (i, j, 0)),
        out_shape=jax.ShapeDtypeStruct((b, s * nsample, c_in), jnp.float32),
    )(idx_col, pfeat)
    return out.reshape(b, s, nsample, c_in)


def _fps_xla(xyz, npoint):
    b, n, _ = xyz.shape

    def step(st, _):
        distance, farthest = st
        centroid = jnp.take_along_axis(xyz, farthest[:, None, None], axis=1)
        d = jnp.sum((xyz - centroid) ** 2, axis=-1)
        distance = jnp.minimum(distance, d)
        nxt = jnp.argmax(distance, axis=-1).astype(jnp.int32)
        return (distance, nxt), farthest

    init = (jnp.full((b, n), 1e10, dtype=xyz.dtype), jnp.zeros((b,), jnp.int32))
    _, idxs = lax.scan(step, init, None, length=npoint)
    fidx = jnp.transpose(idxs, (1, 0))
    return _index_points(xyz, fidx)


def _index_points(points, idx):
    def g(p, i):
        return p[i]

    return jax.vmap(g)(points, idx)


def _ball_query(radius, nsample, xyz, new_xyz):
    b, n, _ = xyz.shape
    s = new_xyz.shape[1]
    sqr = jnp.sum((new_xyz[:, :, None, :] - xyz[:, None, :, :]) ** 2, axis=-1)
    idx = jnp.broadcast_to(jnp.arange(n, dtype=jnp.int32), (b, s, n))
    idx = jnp.where(sqr > radius * radius, n, idx)
    idx = jnp.sort(idx, axis=-1)[:, :, :nsample]
    first = idx[:, :, :1]
    idx = jnp.where(idx == n, jnp.broadcast_to(first, idx.shape), idx)
    return idx


def _mlp(x, layers):
    for (w, g, bta) in layers:
        x = jnp.einsum('...c,cd->...d', x, w)
        m = jnp.mean(x, axis=tuple(range(x.ndim - 1)), keepdims=True)
        v = jnp.var(x, axis=tuple(range(x.ndim - 1)), keepdims=True)
        x = (x - m) / jnp.sqrt(v + 1e-5) * g + bta
        x = jax.nn.relu(x)
    return x


def _sa(xyz, feats, npoint, radius, nsample, layers):
    if npoint is None:
        grouped = xyz[:, None, :, :]
        if feats is not None:
            grouped = jnp.concatenate([grouped, feats[:, None, :, :]], axis=-1)
        new_xyz = jnp.zeros((xyz.shape[0], 1, 3), dtype=xyz.dtype)
    else:
        new_xyz = _fps_pallas(xyz, npoint)
        gidx = _ball_query(radius, nsample, xyz, new_xyz)
        grouped = _index_points(xyz, gidx) - new_xyz[:, :, None, :]
        grouped = jnp.concatenate(
            [grouped, _index_points(feats, gidx)], axis=-1)
    out = _mlp(grouped, layers)
    return new_xyz, jnp.max(out, axis=2)


def kernel(pointcloud, params):
    xyz = pointcloud[..., :3]
    feats = pointcloud[..., 3:]
    xyz, feats = _sa(xyz, feats, 512, 0.2, 64, params['sa1'])
    xyz, feats = _sa(xyz, feats, 128, 0.4, 64, params['sa2'])
    xyz, feats = _sa(xyz, feats, None, None, None, params['sa3'])
    x = feats[:, 0, :]
    w1, g1, b1, w2, g2, b2, w3, b3 = params['fc']
    x = x @ w1
    m = jnp.mean(x, axis=0, keepdims=True)
    v = jnp.var(x, axis=0, keepdims=True)
    x = jax.nn.relu((x - m) / jnp.sqrt(v + 1e-5) * g1 + b1)
    x = x @ w2
    m = jnp.mean(x, axis=0, keepdims=True)
    v = jnp.var(x, axis=0, keepdims=True)
    x = jax.nn.relu((x - m) / jnp.sqrt(v + 1e-5) * g2 + b2)
    return x @ w3 + b3
